# baseline (speedup 1.0000x reference)
import functools

import jax
import jax.numpy as jnp
from jax import lax
from jax.experimental import pallas as pl
from jax.experimental.pallas import tpu as pltpu
from jax.experimental.pallas import tpu_sc as plsc


def _sc_gather(user_table, item_table, uid, pid, nid, s_off, BS, NCAND, D):
    NW = 32
    NNEG = NCAND - 1
    nup = BS // 2
    pu_per_w = nup // NW
    npairs = (BS * NNEG) // 2
    p_per_w = npairs // NW
    CHUNK = p_per_w // 8
    n_chunks = p_per_w // CHUNK
    UB = s_off
    NB = s_off * NNEG

    mesh = plsc.VectorSubcoreMesh(core_axis_name="c", subcore_axis_name="s")

    @functools.partial(
        pl.kernel,
        out_type=(
            jax.ShapeDtypeStruct((nup, 2 * D), jnp.float32),
            jax.ShapeDtypeStruct((nup, 2 * D), jnp.float32),
            jax.ShapeDtypeStruct((npairs, 2 * D), jnp.float32),
        ),
        mesh=mesh,
        scratch_types=[
            pltpu.VMEM((pu_per_w,), jnp.int32),
            pltpu.VMEM((pu_per_w,), jnp.int32),
            pltpu.VMEM((pu_per_w, D), jnp.float32),
            pltpu.VMEM((pu_per_w, D), jnp.float32),
            pltpu.VMEM((pu_per_w,), jnp.int32),
            pltpu.VMEM((pu_per_w,), jnp.int32),
            pltpu.VMEM((pu_per_w, D), jnp.float32),
            pltpu.VMEM((pu_per_w, D), jnp.float32),
            pltpu.VMEM((CHUNK,), jnp.int32),
            pltpu.VMEM((CHUNK,), jnp.int32),
            pltpu.VMEM((CHUNK, D), jnp.float32),
            pltpu.VMEM((CHUNK, D), jnp.float32),
            pltpu.VMEM((CHUNK,), jnp.int32),
            pltpu.VMEM((CHUNK,), jnp.int32),
            pltpu.VMEM((CHUNK, D), jnp.float32),
            pltpu.VMEM((CHUNK, D), jnp.float32),
            pltpu.SemaphoreType.DMA,
            pltpu.SemaphoreType.DMA,
            pltpu.SemaphoreType.DMA,
            pltpu.SemaphoreType.DMA,
            pltpu.SemaphoreType.DMA,
            pltpu.SemaphoreType.DMA,
            pltpu.SemaphoreType.DMA,
            pltpu.SemaphoreType.DMA,
        ],
        compiler_params=pltpu.CompilerParams(use_tc_tiling_on_sc=False),
    )
    def k(ut, it, uid_h, pid_h, nid_h, uout, pout, nout,
          uide, uido, ubufe, ubufo, pide, pido, pbufe, pbufo,
          ie0, io0, bufe0, bufo0, ie1, io1, bufe1, bufo1,
          useme, usemo, pseme, psemo, se0, so0, se1, so1):
        wid = lax.axis_index("s") * 2 + lax.axis_index("c")
        pubase = wid * pu_per_w
        pbase = wid * p_per_w
        ies = (ie0, ie1)
        ios = (io0, io1)
        bufes = (bufe0, bufe1)
        bufos = (bufo0, bufo1)
        sems = ((se0, so0), (se1, so1))

        def issue(j, slot):
            off = pbase + j * CHUNK
            pltpu.sync_copy(nid_h.at[pl.ds(NB + off, CHUNK)], ies[slot])
            pltpu.sync_copy(nid_h.at[pl.ds(NB + npairs + off, CHUNK)], ios[slot])
            ge = pltpu.async_copy(it.at[ies[slot]], bufes[slot], sems[slot][0])
            go = pltpu.async_copy(it.at[ios[slot]], bufos[slot], sems[slot][1])
            return ge, go

        pltpu.sync_copy(uid_h.at[pl.ds(UB + pubase, pu_per_w)], uide)
        pltpu.sync_copy(uid_h.at[pl.ds(UB + nup + pubase, pu_per_w)], uido)
        pltpu.sync_copy(pid_h.at[pl.ds(UB + pubase, pu_per_w)], pide)
        pltpu.sync_copy(pid_h.at[pl.ds(UB + nup + pubase, pu_per_w)], pido)
        uge = pltpu.async_copy(ut.at[uide], ubufe, useme)
        ugo = pltpu.async_copy(ut.at[uido], ubufo, usemo)
        pge = pltpu.async_copy(it.at[pide], pbufe, pseme)
        pgo = pltpu.async_copy(it.at[pido], pbufo, psemo)
        pending = issue(0, 0)
        uge.wait()
        ugo.wait()
        pltpu.sync_copy(ubufe, uout.at[pl.ds(pubase, pu_per_w), pl.ds(0, D)])
        pltpu.sync_copy(ubufo, uout.at[pl.ds(pubase, pu_per_w), pl.ds(D, D)])
        pge.wait()
        pgo.wait()
        pltpu.sync_copy(pbufe, pout.at[pl.ds(pubase, pu_per_w), pl.ds(0, D)])
        pltpu.sync_copy(pbufo, pout.at[pl.ds(pubase, pu_per_w), pl.ds(D, D)])
        for j in range(n_chunks):
            if j + 1 < n_chunks:
                nxt = issue(j + 1, (j + 1) % 2)
            pending[0].wait()
            pending[1].wait()
            off = pbase + j * CHUNK
            pltpu.sync_copy(
                bufes[j % 2], nout.at[pl.ds(off, CHUNK), pl.ds(0, D)]
            )
            pltpu.sync_copy(
                bufos[j % 2], nout.at[pl.ds(off, CHUNK), pl.ds(D, D)]
            )
            if j + 1 < n_chunks:
                pending = nxt

    return k(user_table, item_table, uid, pid, nid)


def _mlp_body(acc_ref, u2_ref, p2_ref, n2_ref, w1u2_ref, w1i2_ref, b12_ref,
              w2_ref, b2_ref, out_ref, *, bp, ncand, d):
    del acc_ref
    a2 = jnp.dot(u2_ref[...], w1u2_ref[...],
                 preferred_element_type=jnp.float32) + b12_ref[...]
    hp = jnp.tanh(
        jnp.dot(p2_ref[...], w1i2_ref[...],
                preferred_element_type=jnp.float32) + a2)
    tn = jnp.dot(n2_ref[...], w1i2_ref[...],
                 preferred_element_type=jnp.float32)
    hn = jnp.tanh(tn.reshape(bp, ncand - 1, 2 * d) + a2[:, None, :])
    w2 = w2_ref[0]
    pe = jnp.sum(hp[:, :d] * w2[None, :], axis=-1, keepdims=True)
    po = jnp.sum(hp[:, d:] * w2[None, :], axis=-1, keepdims=True)
    ne = jnp.sum(hn[:, :, :d] * w2[None, None, :], axis=-1)
    no = jnp.sum(hn[:, :, d:] * w2[None, None, :], axis=-1)
    b2v = b2_ref[0, 0]
    out_ref[0, 0] = jax.nn.sigmoid(jnp.concatenate([pe, ne], axis=1) + b2v)
    out_ref[0, 1] = jax.nn.sigmoid(jnp.concatenate([po, no], axis=1) + b2v)


def _tc_mlp(acc, u2, p2, n2, W1u2, W1i2, b12, w2row, b2, s, NSLICE, BS,
            NCAND, D, interpret=False):
    BP = 256
    grid = (BS // 2 // BP,)
    body = functools.partial(_mlp_body, bp=BP, ncand=NCAND, d=D)
    return pl.pallas_call(
        body,
        grid=grid,
        in_specs=[
            pl.BlockSpec((1, 2, BP, NCAND), lambda i, s=s: (s, 0, i, 0)),
            pl.BlockSpec((BP, 2 * D), lambda i: (i, 0)),
            pl.BlockSpec((BP, 2 * D), lambda i: (i, 0)),
            pl.BlockSpec((BP * (NCAND - 1), 2 * D), lambda i: (i, 0)),
            pl.BlockSpec((2 * D, 2 * D), lambda i: (0, 0)),
            pl.BlockSpec((2 * D, 2 * D), lambda i: (0, 0)),
            pl.BlockSpec((1, 2 * D), lambda i: (0, 0)),
            pl.BlockSpec((1, D), lambda i: (0, 0)),
            pl.BlockSpec((1, 1), lambda i: (0, 0)),
        ],
        out_specs=pl.BlockSpec((1, 2, BP, NCAND), lambda i, s=s: (s, 0, i, 0)),
        out_shape=jax.ShapeDtypeStruct((NSLICE, 2, BS // 2, NCAND),
                                       jnp.float32),
        input_output_aliases={0: 0},
        interpret=interpret,
    )(acc, u2, p2, n2, W1u2, W1i2, b12, w2row, b2.reshape(1, 1))


def kernel(user_id, item_id, neg_items, user_table, item_table, W1, b1, W2, b2):
    B = user_id.shape[0]
    NCAND = neg_items.shape[1] + 1
    D = user_table.shape[1]
    uid = user_id.astype(jnp.int32)
    pid = item_id.astype(jnp.int32)
    nid = neg_items.reshape(-1).astype(jnp.int32)
    W1u = W1[:D, :]
    W1i = W1[D:, :]
    zero = jnp.zeros((D, D), jnp.float32)
    W1u2 = jnp.concatenate(
        [jnp.concatenate([W1u, zero], axis=1),
         jnp.concatenate([zero, W1u], axis=1)], axis=0)
    W1i2 = jnp.concatenate(
        [jnp.concatenate([W1i, zero], axis=1),
         jnp.concatenate([zero, W1i], axis=1)], axis=0)
    b12 = jnp.concatenate([b1, b1]).reshape(1, 2 * D)
    w2row = W2.reshape(1, D)
    NSLICE = 4
    BS = B // NSLICE
    acc = jnp.zeros((NSLICE, 2, BS // 2, NCAND), jnp.float32)
    for s in range(NSLICE):
        u2, p2, n2 = _sc_gather(user_table, item_table, uid, pid, nid,
                                s * BS, BS, NCAND, D)
        acc = _tc_mlp(acc, u2, p2, n2, W1u2, W1i2, b12, w2row, b2,
                      s, NSLICE, BS, NCAND, D)
    return acc.reshape(B, NCAND)
